# speedup vs baseline: 9.2924x; 9.2924x over previous
import jax
import jax.numpy as jnp
from jax.experimental import pallas as pl
from jax.experimental.pallas import tpu as pltpu

_SEARCH = 5
_KNN = 9
_OC = 32
_R = 8


def _knn_body(H, W, CW, x_ref, mf_ref, kv_ref, ix_ref, ug_ref, bm_ref, pad_ref):
    h = pl.program_id(1)
    r0 = h * _R
    pad_ref[...] = jnp.zeros_like(pad_ref)
    for i in range(_R + 4):
        src = r0 - 2 + i

        @pl.when((src >= 0) & (src < H))
        def _(i=i, src=src):
            xr = x_ref[0, :, pl.ds(src, 1), :]
            mr = mf_ref[0, 0, pl.ds(src, 1), :]
            pad_ref[:, i, 2:2 + W] = (xr * (1.0 - mr))[:, 0, :]

    bm = jnp.float32(-jnp.inf)
    for j0 in range(0, W, CW):
        cx = [x_ref[0, c, pl.ds(r0, _R), pl.ds(j0, CW)] for c in (1, 2, 3)]
        dist = []
        nrng = []
        for k in range(_SEARCH * _SEARCH):
            ki, kj = divmod(k, _SEARCH)
            dx = cx[0] - pad_ref[1, ki:ki + _R, kj + j0:kj + j0 + CW]
            dy = cx[1] - pad_ref[2, ki:ki + _R, kj + j0:kj + j0 + CW]
            dz = cx[2] - pad_ref[3, ki:ki + _R, kj + j0:kj + j0 + CW]
            s2 = (dx * dx + dy * dy) + dz * dz
            dist.append(jnp.sqrt(s2 + 1e-12))
            nrng.append(pad_ref[0, ki:ki + _R, kj + j0:kj + j0 + CW])

        inf = jnp.float32(jnp.inf)
        for s in range(_KNN):
            m = dist[0]
            for k in range(1, 25):
                m = jnp.minimum(m, dist[k])
            kv_ref[0, s, :, j0:j0 + CW] = m
            idxf = jnp.zeros_like(m)
            upsel = nrng[0]
            nf = None
            for k in range(25):
                eq = dist[k] == m
                if k == 0:
                    take = eq
                    nf = ~eq
                else:
                    take = eq & nf
                    nf = nf & (~eq)
                    idxf = jnp.where(take, jnp.float32(k), idxf)
                    upsel = jnp.where(take, nrng[k], upsel)
                if s < _KNN - 1:
                    dist[k] = jnp.where(take, inf, dist[k])
            ix_ref[0, s, :, j0:j0 + CW] = idxf
            ug_ref[0, s, :, j0:j0 + CW] = upsel
            bm = jnp.maximum(bm, jnp.max(upsel))
    bm_ref[0, 0] = jnp.broadcast_to(bm, (8, 128))


def _mm_body(W, kv_ref, ix_ref, ug_ref, bm_ref, w_ref, o_ref):
    gmax = jnp.max(bm_ref[...])
    g0 = ug_ref[0, 0]
    cut = 0.05 * jnp.minimum(jnp.maximum(g0, 1.0), gmax)
    feats = []
    for s in range(_KNN):
        feats.append(jnp.where(kv_ref[0, s] > cut, 0.0, ug_ref[0, s]))
    for s in range(_KNN):
        feats.append(ix_ref[0, s])
    f = jnp.stack(feats).reshape(2 * _KNN, _R * W)
    res = jax.lax.dot_general(
        w_ref[...], f, (((1,), (0,)), ((), ())),
        preferred_element_type=jnp.float32,
        precision=jax.lax.Precision.HIGHEST)
    res = jnp.where(res >= 0, res, 0.01 * res)
    o_ref[0] = res.reshape(_OC, _R, W)


def kernel(x, binary_mask, range_weight):
    B, C, H, W = x.shape
    HB = H // _R
    CW = min(512, W)
    mf = binary_mask.astype(jnp.float32)
    w = range_weight.reshape(_OC, 2 * _KNN)

    f32 = jnp.float32
    kv, ix, ug, bm = pl.pallas_call(
        lambda *refs: _knn_body(H, W, CW, *refs),
        grid=(B, HB),
        in_specs=[
            pl.BlockSpec((1, C, H, W), lambda b, h: (b, 0, 0, 0)),
            pl.BlockSpec((1, 1, H, W), lambda b, h: (b, 0, 0, 0)),
        ],
        out_specs=[
            pl.BlockSpec((1, _KNN, _R, W), lambda b, h: (b, 0, h, 0)),
            pl.BlockSpec((1, _KNN, _R, W), lambda b, h: (b, 0, h, 0)),
            pl.BlockSpec((1, _KNN, _R, W), lambda b, h: (b, 0, h, 0)),
            pl.BlockSpec((1, 1, 8, 128), lambda b, h: (b, h, 0, 0)),
        ],
        out_shape=[
            jax.ShapeDtypeStruct((B, _KNN, H, W), f32),
            jax.ShapeDtypeStruct((B, _KNN, H, W), f32),
            jax.ShapeDtypeStruct((B, _KNN, H, W), f32),
            jax.ShapeDtypeStruct((B, HB, 8, 128), f32),
        ],
        scratch_shapes=[pltpu.VMEM((C, _R + 4, W + 4), f32)],
    )(x, mf)

    out = pl.pallas_call(
        lambda *refs: _mm_body(W, *refs),
        grid=(B, HB),
        in_specs=[
            pl.BlockSpec((1, _KNN, _R, W), lambda b, h: (b, 0, h, 0)),
            pl.BlockSpec((1, _KNN, _R, W), lambda b, h: (b, 0, h, 0)),
            pl.BlockSpec((1, _KNN, _R, W), lambda b, h: (b, 0, h, 0)),
            pl.BlockSpec((B, HB, 8, 128), lambda b, h: (0, 0, 0, 0)),
            pl.BlockSpec((_OC, 2 * _KNN), lambda b, h: (0, 0)),
        ],
        out_specs=pl.BlockSpec((1, _OC, _R, W), lambda b, h: (b, 0, h, 0)),
        out_shape=jax.ShapeDtypeStruct((B, _OC, H, W), f32),
    )(kv, ix, ug, bm, w)

    return out, kv


# speedup vs baseline: 10.8438x; 1.1670x over previous
import jax
import jax.numpy as jnp
from jax.experimental import pallas as pl
from jax.experimental.pallas import tpu as pltpu

_SEARCH = 5
_KNN = 9
_OC = 32
_R = 8
_CW = 256


def _tree(vals, op):
    vals = list(vals)
    while len(vals) > 1:
        nxt = [op(vals[i], vals[i + 1]) for i in range(0, len(vals) - 1, 2)]
        if len(vals) % 2:
            nxt.append(vals[-1])
        vals = nxt
    return vals[0]


def _knn_body(H, W, CW, x_ref, mf_ref, kv_ref, ix_ref, ug_ref, bm_ref, pad_ref):
    h = pl.program_id(1)
    r0 = h * _R
    pad_ref[...] = jnp.zeros_like(pad_ref)
    for i in range(_R + 4):
        src = r0 - 2 + i

        @pl.when((src >= 0) & (src < H))
        def _(i=i, src=src):
            xr = x_ref[0, :, pl.ds(src, 1), :]
            mr = mf_ref[0, 0, pl.ds(src, 1), :]
            pad_ref[:, i, 2:2 + W] = (xr * (1.0 - mr))[:, 0, :]

    bm = jnp.float32(-jnp.inf)
    for j0 in range(0, W, CW):
        cx = [x_ref[0, c, pl.ds(r0, _R), pl.ds(j0, CW)] for c in (1, 2, 3)]
        dist = []
        nrng = []
        for k in range(_SEARCH * _SEARCH):
            ki, kj = divmod(k, _SEARCH)
            dx = cx[0] - pad_ref[1, ki:ki + _R, kj + j0:kj + j0 + CW]
            dy = cx[1] - pad_ref[2, ki:ki + _R, kj + j0:kj + j0 + CW]
            dz = cx[2] - pad_ref[3, ki:ki + _R, kj + j0:kj + j0 + CW]
            s2 = (dx * dx + dy * dy) + dz * dz
            dist.append(jnp.sqrt(s2 + 1e-12))
            nrng.append(pad_ref[0, ki:ki + _R, kj + j0:kj + j0 + CW])

        inf = jnp.float32(jnp.inf)
        big = jnp.int32(127)
        for s in range(_KNN):
            m = _tree(dist, jnp.minimum)
            kv_ref[0, s, :, j0:j0 + CW] = m
            cands = [jnp.where(dist[k] == m, jnp.int32(k), big)
                     for k in range(25)]
            j = _tree(cands, jnp.minimum)
            cond = [j == jnp.int32(k) for k in range(25)]
            upsel = _tree([jnp.where(cond[k], nrng[k], 0.0)
                           for k in range(25)], jnp.add)
            if s < _KNN - 1:
                dist = [jnp.where(cond[k], inf, dist[k]) for k in range(25)]
            ix_ref[0, s, :, j0:j0 + CW] = j.astype(jnp.float32)
            ug_ref[0, s, :, j0:j0 + CW] = upsel
            bm = jnp.maximum(bm, jnp.max(upsel))
    bm_ref[0, 0] = jnp.broadcast_to(bm, (8, 128))


def _mm_body(W, kv_ref, ix_ref, ug_ref, bm_ref, w_ref, o_ref):
    gmax = jnp.max(bm_ref[...])
    g0 = ug_ref[0, 0]
    cut = 0.05 * jnp.minimum(jnp.maximum(g0, 1.0), gmax)
    feats = []
    for s in range(_KNN):
        feats.append(jnp.where(kv_ref[0, s] > cut, 0.0, ug_ref[0, s]))
    for s in range(_KNN):
        feats.append(ix_ref[0, s])
    f = jnp.stack(feats).reshape(2 * _KNN, _R * W)
    res = jax.lax.dot_general(
        w_ref[...], f, (((1,), (0,)), ((), ())),
        preferred_element_type=jnp.float32,
        precision=jax.lax.Precision.HIGHEST)
    res = jnp.where(res >= 0, res, 0.01 * res)
    o_ref[0] = res.reshape(_OC, _R, W)


def kernel(x, binary_mask, range_weight):
    B, C, H, W = x.shape
    HB = H // _R
    CW = min(_CW, W)
    mf = binary_mask.astype(jnp.float32)
    w = range_weight.reshape(_OC, 2 * _KNN)

    f32 = jnp.float32
    kv, ix, ug, bm = pl.pallas_call(
        lambda *refs: _knn_body(H, W, CW, *refs),
        grid=(B, HB),
        in_specs=[
            pl.BlockSpec((1, C, H, W), lambda b, h: (b, 0, 0, 0)),
            pl.BlockSpec((1, 1, H, W), lambda b, h: (b, 0, 0, 0)),
        ],
        out_specs=[
            pl.BlockSpec((1, _KNN, _R, W), lambda b, h: (b, 0, h, 0)),
            pl.BlockSpec((1, _KNN, _R, W), lambda b, h: (b, 0, h, 0)),
            pl.BlockSpec((1, _KNN, _R, W), lambda b, h: (b, 0, h, 0)),
            pl.BlockSpec((1, 1, 8, 128), lambda b, h: (b, h, 0, 0)),
        ],
        out_shape=[
            jax.ShapeDtypeStruct((B, _KNN, H, W), f32),
            jax.ShapeDtypeStruct((B, _KNN, H, W), f32),
            jax.ShapeDtypeStruct((B, _KNN, H, W), f32),
            jax.ShapeDtypeStruct((B, HB, 8, 128), f32),
        ],
        scratch_shapes=[pltpu.VMEM((C, _R + 4, W + 4), f32)],
    )(x, mf)

    out = pl.pallas_call(
        lambda *refs: _mm_body(W, *refs),
        grid=(B, HB),
        in_specs=[
            pl.BlockSpec((1, _KNN, _R, W), lambda b, h: (b, 0, h, 0)),
            pl.BlockSpec((1, _KNN, _R, W), lambda b, h: (b, 0, h, 0)),
            pl.BlockSpec((1, _KNN, _R, W), lambda b, h: (b, 0, h, 0)),
            pl.BlockSpec((B, HB, 8, 128), lambda b, h: (0, 0, 0, 0)),
            pl.BlockSpec((_OC, 2 * _KNN), lambda b, h: (0, 0)),
        ],
        out_specs=pl.BlockSpec((1, _OC, _R, W), lambda b, h: (b, 0, h, 0)),
        out_shape=jax.ShapeDtypeStruct((B, _OC, H, W), f32),
    )(kv, ix, ug, bm, w)

    return out, kv
